# baseline (speedup 1.0000x reference)
import jax
import jax.numpy as jnp
from jax.experimental import pallas as pl
from jax.experimental.pallas import tpu as pltpu

_CH = 200
_NBUF = 5
_SUP = _CH * _NBUF


def _gcn_kernel(x_ref, w_ref, b_ref, adj_ref, support_ref, hidden_ref,
                out_ref, bufs, sems):
    i = pl.program_id(0)
    nsteps = pl.num_programs(0)

    def start(chunk, slot):
        pltpu.make_async_copy(
            adj_ref.at[pl.ds(chunk * _CH, _CH), :],
            bufs.at[slot],
            sems.at[slot],
        ).start()

    @pl.when(i == 0)
    def _():
        for s in range(_NBUF):
            start(s, s)
        support_ref[...] = jnp.dot(
            x_ref[...], w_ref[...], preferred_element_type=jnp.float32
        )

    for s in range(_NBUF):
        pltpu.make_async_copy(
            adj_ref.at[pl.ds((i * _NBUF + s) * _CH, _CH), :],
            bufs.at[s],
            sems.at[s],
        ).wait()
        acc = jnp.dot(
            bufs[s], support_ref[...], preferred_element_type=jnp.float32
        )
        acc = acc + b_ref[...]
        out_ref[pl.ds(s * _CH, _CH), :] = acc
        hidden_ref[pl.ds(s * _CH, _CH), :] = jnp.maximum(acc, 0.0)

        @pl.when(i + 1 < nsteps)
        def _():
            start((i + 1) * _NBUF + s, s)


def kernel(x, adj, grad_adj, W, b):
    N, din = x.shape
    dout = W.shape[1]
    grid = (N // _SUP,)

    b2 = b.reshape(1, dout)

    support, hidden, output = pl.pallas_call(
        _gcn_kernel,
        grid=grid,
        in_specs=[
            pl.BlockSpec((N, din), lambda i: (0, 0)),
            pl.BlockSpec((din, dout), lambda i: (0, 0)),
            pl.BlockSpec((1, dout), lambda i: (0, 0)),
            pl.BlockSpec(memory_space=pl.ANY),
        ],
        out_specs=[
            pl.BlockSpec((N, dout), lambda i: (0, 0)),
            pl.BlockSpec((_SUP, dout), lambda i: (i, 0)),
            pl.BlockSpec((_SUP, dout), lambda i: (i, 0)),
        ],
        out_shape=[
            jax.ShapeDtypeStruct((N, dout), jnp.float32),
            jax.ShapeDtypeStruct((N, dout), jnp.float32),
            jax.ShapeDtypeStruct((N, dout), jnp.float32),
        ],
        scratch_shapes=[
            pltpu.VMEM((_NBUF, _CH, N), jnp.float32),
            pltpu.SemaphoreType.DMA((_NBUF,)),
        ],
    )(x, W, b2, adj)

    return (support, hidden, output)


# speedup vs baseline: 1.0166x; 1.0166x over previous
import jax
import jax.numpy as jnp
from jax.experimental import pallas as pl

_BR = 200


def _gcn_kernel(x_ref, w_ref, b_ref, adjt_ref, adjb_ref, support_ref,
                hidden_ref, out_ref):
    i = pl.program_id(0)

    @pl.when(i == 0)
    def _():
        support_ref[...] = jnp.dot(
            x_ref[...], w_ref[...], preferred_element_type=jnp.float32
        )

    acc_t = jnp.dot(
        adjt_ref[0], support_ref[...], preferred_element_type=jnp.float32
    ) + b_ref[...]
    acc_b = jnp.dot(
        adjb_ref[0], support_ref[...], preferred_element_type=jnp.float32
    ) + b_ref[...]
    out_ref[0] = acc_t
    out_ref[1] = acc_b
    hidden_ref[0] = jnp.maximum(acc_t, 0.0)
    hidden_ref[1] = jnp.maximum(acc_b, 0.0)


def kernel(x, adj, grad_adj, W, b):
    N, din = x.shape
    dout = W.shape[1]
    half = N // 2
    grid = (half // _BR,)

    b2 = b.reshape(1, dout)
    adj3 = adj.reshape(2, half, N)

    support, hidden3, output3 = pl.pallas_call(
        _gcn_kernel,
        grid=grid,
        in_specs=[
            pl.BlockSpec((N, din), lambda i: (0, 0)),
            pl.BlockSpec((din, dout), lambda i: (0, 0)),
            pl.BlockSpec((1, dout), lambda i: (0, 0)),
            pl.BlockSpec((1, _BR, N), lambda i: (0, i, 0)),
            pl.BlockSpec((1, _BR, N), lambda i: (1, i, 0)),
        ],
        out_specs=[
            pl.BlockSpec((N, dout), lambda i: (0, 0)),
            pl.BlockSpec((2, _BR, dout), lambda i: (0, i, 0)),
            pl.BlockSpec((2, _BR, dout), lambda i: (0, i, 0)),
        ],
        out_shape=[
            jax.ShapeDtypeStruct((N, dout), jnp.float32),
            jax.ShapeDtypeStruct((2, half, dout), jnp.float32),
            jax.ShapeDtypeStruct((2, half, dout), jnp.float32),
        ],
    )(x, W, b2, adj3, adj3)

    return (support, hidden3.reshape(N, dout), output3.reshape(N, dout))


# speedup vs baseline: 1.0278x; 1.0111x over previous
import jax
import jax.numpy as jnp
from jax.experimental import pallas as pl


def _gcn_kernel(x_ref, w_ref, b_ref, adj_ref, support_ref, hidden_ref, out_ref):
    i = pl.program_id(0)

    @pl.when(i == 0)
    def _():
        support_ref[...] = jnp.dot(
            x_ref[...], w_ref[...], preferred_element_type=jnp.float32
        )

    acc = jnp.dot(
        adj_ref[...], support_ref[...], preferred_element_type=jnp.float32
    )
    acc = acc + b_ref[...]
    out_ref[...] = acc
    hidden_ref[...] = jnp.maximum(acc, 0.0)


def kernel(x, adj, grad_adj, W, b):
    N, din = x.shape
    dout = W.shape[1]

    block_r = 400
    if N % block_r != 0:
        block_r = N
    grid = (N // block_r,)

    b2 = b.reshape(1, dout)

    support, hidden, output = pl.pallas_call(
        _gcn_kernel,
        grid=grid,
        in_specs=[
            pl.BlockSpec((N, din), lambda i: (0, 0)),
            pl.BlockSpec((din, dout), lambda i: (0, 0)),
            pl.BlockSpec((1, dout), lambda i: (0, 0)),
            pl.BlockSpec((block_r, N), lambda i: (i, 0)),
        ],
        out_specs=[
            pl.BlockSpec((N, dout), lambda i: (0, 0)),
            pl.BlockSpec((block_r, dout), lambda i: (i, 0)),
            pl.BlockSpec((block_r, dout), lambda i: (i, 0)),
        ],
        out_shape=[
            jax.ShapeDtypeStruct((N, dout), jnp.float32),
            jax.ShapeDtypeStruct((N, dout), jnp.float32),
            jax.ShapeDtypeStruct((N, dout), jnp.float32),
        ],
    )(x, W, b2, adj)

    return (support, hidden, output)
